# baseline (speedup 1.0000x reference)
import functools

import jax
import jax.numpy as jnp
from jax import lax
from jax.experimental import pallas as pl
from jax.experimental.pallas import tpu as pltpu

_NSPLIT = 2
_CHUNK = 2000


def _argmax_row_chunk(xc, k, iota_ck, iota_row):
    mx = jnp.max(xc, axis=1, keepdims=True)
    first = jnp.min(jnp.where(xc == mx, iota_ck, float(k)), axis=1, keepdims=True)
    onehot = (iota_ck == first).astype(jnp.float32)
    return lax.dot_general(
        iota_row, onehot, (((1,), (1,)), ((), ())),
        preferred_element_type=jnp.float32,
    )


def _assign_body(*refs):
    mbt_ref = refs[0]
    x_refs = refs[1:-2]
    ha_ref, comp_ref = refs[-2:]
    k = x_refs[0].shape[2]
    bn = x_refs[0].shape[1]
    nch = bn // _CHUNK
    iota_ck = lax.broadcasted_iota(jnp.int32, (_CHUNK, k), 1).astype(jnp.float32)
    iota_row = lax.broadcasted_iota(jnp.int32, (1, k), 1).astype(jnp.float32)
    iota_seg = lax.broadcasted_iota(jnp.int32, (k, _CHUNK), 0).astype(jnp.float32)
    acc = [None] * 4
    for s, x_ref in enumerate(x_refs):
        for j in range(nch):
            xc = x_ref[0, pl.ds(j * _CHUNK, _CHUNK), :]
            rowj = _argmax_row_chunk(xc, k, iota_ck, iota_row)
            ha_ref[0, s * nch + j, :] = rowj[0].astype(jnp.int32)
            maskj = iota_seg == rowj
            for cc in range(4):
                sgn = 1.0 if cc < 2 else -1.0
                fill = 1.0 if cc < 2 else 0.0
                v = mbt_ref[0, cc : cc + 1, s * nch + j, :] * sgn
                contrib = jnp.min(jnp.where(maskj, v, fill), axis=1, keepdims=True)
                acc[cc] = contrib if acc[cc] is None else jnp.minimum(acc[cc], contrib)
    comp_ref[0] = jnp.concatenate(acc, axis=1)


def _assign_call(micro_boxes_t, x, nsplit):
    b, n, k = x.shape
    bn = n // nsplit
    nrows = n // _CHUNK
    in_specs = [pl.BlockSpec((1, 4, nrows, _CHUNK), lambda bb: (bb, 0, 0, 0))] + [
        pl.BlockSpec((1, bn, k), functools.partial(lambda s, bb: (bb, s, 0), s))
        for s in range(nsplit)
    ]
    return pl.pallas_call(
        _assign_body,
        grid=(b,),
        in_specs=in_specs,
        out_specs=[
            pl.BlockSpec((1, nrows, _CHUNK), lambda bb: (bb, 0, 0)),
            pl.BlockSpec((1, k, 4), lambda bb: (bb, 0, 0)),
        ],
        out_shape=[
            jax.ShapeDtypeStruct((b, nrows, _CHUNK), jnp.int32),
            jax.ShapeDtypeStruct((b, k, 4), jnp.float32),
        ],
        compiler_params=pltpu.CompilerParams(
            dimension_semantics=("arbitrary",),
        ),
    )(micro_boxes_t, *([x] * nsplit))


def _class_body(*refs):
    o_ref = refs[-1]
    x_refs = refs[:-1]
    bn, k = x_refs[0].shape[1], x_refs[0].shape[2]
    nch = bn // _CHUNK
    iota_ck = lax.broadcasted_iota(jnp.int32, (_CHUNK, k), 1).astype(jnp.float32)
    iota_row = lax.broadcasted_iota(jnp.int32, (1, k), 1).astype(jnp.float32)
    for s, x_ref in enumerate(x_refs):
        for j in range(nch):
            xc = x_ref[0, pl.ds(j * _CHUNK, _CHUNK), :]
            rowj = _argmax_row_chunk(xc, k, iota_ck, iota_row)
            o_ref[0, s * nch + j, :] = rowj[0].astype(jnp.int32)


def _class_call(x, nsplit):
    b, n, k = x.shape
    bn = n // nsplit
    nrows = n // _CHUNK
    in_specs = [
        pl.BlockSpec((1, bn, k), functools.partial(lambda s, bb: (bb, s, 0), s))
        for s in range(nsplit)
    ]
    return pl.pallas_call(
        _class_body,
        grid=(b,),
        in_specs=in_specs,
        out_specs=pl.BlockSpec((1, nrows, _CHUNK), lambda bb: (bb, 0, 0)),
        out_shape=jax.ShapeDtypeStruct((b, nrows, _CHUNK), jnp.int32),
        compiler_params=pltpu.CompilerParams(
            dimension_semantics=("arbitrary",),
        ),
    )(*([x] * nsplit))


def kernel(micro_boxes, assign_probs, class_logits):
    b, n, _ = micro_boxes.shape
    k = assign_probs.shape[-1]
    nrows = n // _CHUNK

    mbt = jnp.transpose(micro_boxes, (0, 2, 1)).reshape(b, 4, nrows, _CHUNK)
    ha, comp = _assign_call(mbt, assign_probs, _NSPLIT)
    pc = _class_call(class_logits, _NSPLIT)

    hard_assign = ha.reshape(b, n)
    pred_classes = pc.reshape(b, n)
    signs = jnp.array([1.0, 1.0, -1.0, -1.0], jnp.float32)
    comp_boxes = comp * signs[None, None, :]
    micro_keep_mask = jnp.ones((b, n), dtype=bool)
    component_ids = jnp.broadcast_to(jnp.arange(k, dtype=jnp.int32), (b, k))
    return (hard_assign, pred_classes, micro_boxes, micro_keep_mask, comp_boxes, component_ids)
